# Initial kernel scaffold; baseline (speedup 1.0000x reference)
import jax
import jax.numpy as jnp
from jax.experimental import pallas as pl


def kernel(x_pfc, x_vtx, batch_pfc, batch_vtx, P1, pb1, P2, pb2, V1, vb1, V2, vb2, C1, cb1, O1, ob1, O2, ob2, O3, ob3, O4, ob4):
    raise NotImplementedError("write your pallas kernel here")


# speedup vs baseline: 2.0549x; 2.0549x over previous
import functools

import jax
import jax.numpy as jnp
from jax.experimental import pallas as pl
from jax.experimental.pallas import tpu as pltpu

HID = 32
K = 16
NB = 16
CHUNK = 512
TILE = 256
BIG = 3e38
MASKVAL = 1e10
IDXBIG = 1e9
HIGHEST = jax.lax.Precision.HIGHEST


def _lr(x):
    return jnp.where(x > 0, x, 0.01 * x)


def _encode_kernel(x_ref, w1_ref, b1_ref, w2_ref, b2_ref, xe_ref):
    h = _lr(jnp.dot(x_ref[...], w1_ref[...],
                    preferred_element_type=jnp.float32) + b1_ref[...])
    xe_ref[...] = _lr(jnp.dot(h, w2_ref[...],
                              preferred_element_type=jnp.float32) + b2_ref[...])


def _encode(x, w1, b1, w2, b2):
    n, din = x.shape
    nc = n // CHUNK
    return pl.pallas_call(
        _encode_kernel,
        grid=(nc,),
        in_specs=[
            pl.BlockSpec((CHUNK, din), lambda r: (r, 0)),
            pl.BlockSpec((din, HID), lambda r: (0, 0)),
            pl.BlockSpec((1, HID), lambda r: (0, 0)),
            pl.BlockSpec((HID, HID), lambda r: (0, 0)),
            pl.BlockSpec((1, HID), lambda r: (0, 0)),
        ],
        out_specs=pl.BlockSpec((CHUNK, HID), lambda r: (r, 0)),
        out_shape=jax.ShapeDtypeStruct((n, HID), jnp.float32),
    )(x, w1, b1, w2, b2)


def _edge_conv(t, et, d_ref, st_ref, s2_ref, xe_ref, est_ref, c1, cb1, nc,
               iota):
    f32 = jnp.float32

    def build(r, c):
        m = jnp.dot(t, st_ref[r], preferred_element_type=f32)
        d = -2.0 * m + s2_ref[r]
        same = jnp.dot(et, est_ref[r], preferred_element_type=f32)
        d_ref[r] = jnp.where(same > 0.5, d, MASKVAL)
        return c

    jax.lax.fori_loop(0, nc, build, 0)

    def one_round(_, acc):
        def scan_chunk(r, mi):
            m, i = mi
            dd = d_ref[r]
            col = iota + jax.lax.convert_element_type(r, f32) * CHUNK
            mr = jnp.min(dd, axis=1, keepdims=True)
            ir = jnp.min(jnp.where(dd == mr, col, IDXBIG), axis=1,
                         keepdims=True)
            tie = mr == m
            i = jnp.where(mr < m, ir, jnp.where(tie, jnp.minimum(i, ir), i))
            m = jnp.minimum(m, mr)
            return m, i

        m0 = jnp.full((TILE, 1), BIG, f32)
        i0 = jnp.full((TILE, 1), IDXBIG, f32)
        _, isel = jax.lax.fori_loop(0, nc, scan_chunk, (m0, i0))

        def gather_chunk(r, g):
            dd = d_ref[r]
            col = iota + jax.lax.convert_element_type(r, f32) * CHUNK
            hit = col == isel
            d_ref[r] = jnp.where(hit, BIG, dd)
            return g + jnp.dot(hit.astype(f32), xe_ref[r],
                               preferred_element_type=f32, precision=HIGHEST)

        xj = jax.lax.fori_loop(0, nc, gather_chunk,
                               jnp.zeros((TILE, HID), f32))
        mm = jnp.concatenate([t, xj - t], axis=1)
        msg = _lr(jnp.dot(mm, c1, preferred_element_type=f32) + cb1)
        return jnp.maximum(acc, msg)

    acc0 = jnp.full((TILE, HID), -BIG, f32)
    return jax.lax.fori_loop(0, K, one_round, acc0)


def _net_kernel(nc1, nc2,
                xe_ref, et_ref,
                st1_ref, s21_ref, xe3_ref, est1_ref,
                st2_ref, s22_ref, xv3_ref, est2_ref,
                c1_ref, cb1_ref,
                o1_ref, ob1_ref, o2_ref, ob2_ref,
                o3_ref, ob3_ref, o4_ref, ob4_ref,
                out_ref, d1_ref, d2_ref):
    f32 = jnp.float32
    iota = jax.lax.broadcasted_iota(jnp.int32, (TILE, CHUNK), 1).astype(f32)
    et = et_ref[...]
    c1 = c1_ref[...]
    cb1 = cb1_ref[...]

    feats1 = _edge_conv(xe_ref[...], et, d1_ref, st1_ref, s21_ref, xe3_ref,
                        est1_ref, c1, cb1, nc1, iota)
    feats2 = _edge_conv(feats1, et, d2_ref, st2_ref, s22_ref, xv3_ref,
                        est2_ref, c1, cb1, nc2, iota)

    h = _lr(jnp.dot(feats2, o1_ref[...], preferred_element_type=f32)
            + ob1_ref[...])
    h = _lr(jnp.dot(h, o2_ref[...], preferred_element_type=f32) + ob2_ref[...])
    h = _lr(jnp.dot(h, o3_ref[...], preferred_element_type=f32) + ob3_ref[...])
    h = _lr(jnp.dot(h, o4_ref[...], preferred_element_type=f32) + ob4_ref[...])
    out_ref[...] = h


def kernel(x_pfc, x_vtx, batch_pfc, batch_vtx,
           P1, pb1, P2, pb2, V1, vb1, V2, vb2, C1, cb1,
           O1, ob1, O2, ob2, O3, ob3, O4, ob4):
    f32 = jnp.float32
    n_pfc = x_pfc.shape[0]
    n_vtx = x_vtx.shape[0]
    nc1 = n_pfc // CHUNK
    nc2 = n_vtx // CHUNK
    ntiles = n_pfc // TILE

    xe = _encode(x_pfc, P1, pb1.reshape(1, -1), P2, pb2.reshape(1, -1))
    xv = _encode(x_vtx, V1, vb1.reshape(1, -1), V2, vb2.reshape(1, -1))

    st1 = xe.T.reshape(HID, nc1, CHUNK).transpose(1, 0, 2)
    s21 = jnp.sum(xe ** 2, axis=1).reshape(nc1, 1, CHUNK)
    xe3 = xe.reshape(nc1, CHUNK, HID)
    st2 = xv.T.reshape(HID, nc2, CHUNK).transpose(1, 0, 2)
    s22 = jnp.sum(xv ** 2, axis=1).reshape(nc2, 1, CHUNK)
    xv3 = xv.reshape(nc2, CHUNK, HID)

    et_pfc = jax.nn.one_hot(batch_pfc, NB, dtype=f32)
    est1 = et_pfc.T.reshape(NB, nc1, CHUNK).transpose(1, 0, 2)
    est2 = (jax.nn.one_hot(batch_vtx, NB, dtype=f32)
            .T.reshape(NB, nc2, CHUNK).transpose(1, 0, 2))

    full3 = lambda a: pl.BlockSpec(a.shape, lambda t: (0, 0, 0))
    full2 = lambda a: pl.BlockSpec(a.shape, lambda t: (0, 0))

    out = pl.pallas_call(
        functools.partial(_net_kernel, nc1, nc2),
        grid=(ntiles,),
        in_specs=[
            pl.BlockSpec((TILE, HID), lambda t: (t, 0)),
            pl.BlockSpec((TILE, NB), lambda t: (t, 0)),
            full3(st1), full3(s21), full3(xe3), full3(est1),
            full3(st2), full3(s22), full3(xv3), full3(est2),
            full2(C1), pl.BlockSpec((1, HID), lambda t: (0, 0)),
            full2(O1), pl.BlockSpec((1, 64), lambda t: (0, 0)),
            full2(O2), pl.BlockSpec((1, 32), lambda t: (0, 0)),
            full2(O3), pl.BlockSpec((1, 4), lambda t: (0, 0)),
            full2(O4), pl.BlockSpec((1, 1), lambda t: (0, 0)),
        ],
        out_specs=pl.BlockSpec((TILE, 1), lambda t: (t, 0)),
        out_shape=jax.ShapeDtypeStruct((n_pfc, 1), f32),
        scratch_shapes=[
            pltpu.VMEM((nc1, TILE, CHUNK), f32),
            pltpu.VMEM((nc2, TILE, CHUNK), f32),
        ],
    )(xe, et_pfc, st1, s21, xe3, est1, st2, s22, xv3, est2,
      C1, cb1.reshape(1, -1),
      O1, ob1.reshape(1, -1), O2, ob2.reshape(1, -1),
      O3, ob3.reshape(1, -1), O4, ob4.reshape(1, -1))
    return (out, batch_pfc)


# speedup vs baseline: 5.1065x; 2.4850x over previous
import functools

import jax
import jax.numpy as jnp
from jax.experimental import pallas as pl
from jax.experimental.pallas import tpu as pltpu

HID = 32
K = 16
NB = 16
CHUNK = 512
TILE = 256
BIG = 3e38
MASKVAL = 1e10
IDXBIG = 1e9
HIGHEST = jax.lax.Precision.HIGHEST


def _lr(x):
    return jnp.where(x > 0, x, 0.01 * x)


def _encode_kernel(x_ref, w1_ref, b1_ref, w2_ref, b2_ref, xe_ref):
    h = _lr(jnp.dot(x_ref[...], w1_ref[...],
                    preferred_element_type=jnp.float32) + b1_ref[...])
    xe_ref[...] = _lr(jnp.dot(h, w2_ref[...],
                              preferred_element_type=jnp.float32) + b2_ref[...])


def _encode(x, w1, b1, w2, b2):
    n, din = x.shape
    nc = n // CHUNK
    return pl.pallas_call(
        _encode_kernel,
        grid=(nc,),
        in_specs=[
            pl.BlockSpec((CHUNK, din), lambda r: (r, 0)),
            pl.BlockSpec((din, HID), lambda r: (0, 0)),
            pl.BlockSpec((1, HID), lambda r: (0, 0)),
            pl.BlockSpec((HID, HID), lambda r: (0, 0)),
            pl.BlockSpec((1, HID), lambda r: (0, 0)),
        ],
        out_specs=pl.BlockSpec((CHUNK, HID), lambda r: (r, 0)),
        out_shape=jax.ShapeDtypeStruct((n, HID), jnp.float32),
    )(x, w1, b1, w2, b2)


def _edge_conv(t, et, d_ref, dfix_ref, st_ref, s2_ref, xe_ref, est_ref,
               c1, cb1, c0, nw, iota):
    f32 = jnp.float32
    base = jax.lax.convert_element_type(c0, f32) * CHUNK

    same0 = jnp.dot(et, est_ref[0], preferred_element_type=f32)
    dfix_ref[...] = jnp.where(same0 > 0.5, BIG, MASKVAL)

    def build(j, c):
        r = c0 + j
        m = jnp.dot(t, st_ref[r], preferred_element_type=f32)
        d = -2.0 * m + s2_ref[r]
        same = jnp.dot(et, est_ref[r], preferred_element_type=f32)
        d_ref[j] = jnp.where(same > 0.5, d, BIG)
        return c

    jax.lax.fori_loop(0, nw, build, 0)

    def one_round(_, acc):
        def scan_chunk(j, mi):
            m, i = mi
            dd = d_ref[j]
            col = iota + base + jax.lax.convert_element_type(j, f32) * CHUNK
            mr = jnp.min(dd, axis=1, keepdims=True)
            ir = jnp.min(jnp.where(dd == mr, col, IDXBIG), axis=1,
                         keepdims=True)
            tie = mr == m
            i = jnp.where(mr < m, ir, jnp.where(tie, jnp.minimum(i, ir), i))
            m = jnp.minimum(m, mr)
            return m, i

        m0 = jnp.full((TILE, 1), BIG, f32)
        i0 = jnp.full((TILE, 1), IDXBIG, f32)
        msel, isel = jax.lax.fori_loop(0, nw, scan_chunk, (m0, i0))
        dd0 = dfix_ref[...]
        mr0 = jnp.min(dd0, axis=1, keepdims=True)
        ir0 = jnp.min(jnp.where(dd0 == mr0, iota, IDXBIG), axis=1,
                      keepdims=True)
        tie0 = mr0 == msel
        isel = jnp.where(mr0 < msel, ir0,
                         jnp.where(tie0, jnp.minimum(isel, ir0), isel))
        msel = jnp.minimum(msel, mr0)
        fromfix = msel == MASKVAL

        def gather_chunk(j, g):
            dd = d_ref[j]
            col = iota + base + jax.lax.convert_element_type(j, f32) * CHUNK
            hit = (col == isel) & jnp.logical_not(fromfix)
            d_ref[j] = jnp.where(hit, BIG, dd)
            return g + jnp.dot(hit.astype(f32), xe_ref[c0 + j],
                               preferred_element_type=f32, precision=HIGHEST)

        xj = jax.lax.fori_loop(0, nw, gather_chunk,
                               jnp.zeros((TILE, HID), f32))
        hit0 = (iota == isel) & fromfix
        dfix_ref[...] = jnp.where(hit0, BIG, dfix_ref[...])
        xj = xj + jnp.dot(hit0.astype(f32), xe_ref[0],
                          preferred_element_type=f32, precision=HIGHEST)

        mm = jnp.concatenate([t, xj - t], axis=1)
        msg = _lr(jnp.dot(mm, c1, preferred_element_type=f32) + cb1)
        return jnp.maximum(acc, msg)

    acc0 = jnp.full((TILE, HID), -BIG, f32)
    return jax.lax.fori_loop(0, K, one_round, acc0)


def _net_kernel(scal_ref,
                xe_ref, et_ref,
                st1_ref, s21_ref, xe3_ref, est1_ref,
                st2_ref, s22_ref, xv3_ref, est2_ref,
                c1_ref, cb1_ref,
                o1_ref, ob1_ref, o2_ref, ob2_ref,
                o3_ref, ob3_ref, o4_ref, ob4_ref,
                out_ref, d_ref, dfix_ref):
    f32 = jnp.float32
    tid = pl.program_id(0)
    iota = jax.lax.broadcasted_iota(jnp.int32, (TILE, CHUNK), 1).astype(f32)
    et = et_ref[...]
    c1 = c1_ref[...]
    cb1 = cb1_ref[...]

    feats1 = _edge_conv(xe_ref[...], et, d_ref, dfix_ref, st1_ref, s21_ref,
                        xe3_ref, est1_ref, c1, cb1,
                        scal_ref[0, tid], scal_ref[1, tid], iota)
    feats2 = _edge_conv(feats1, et, d_ref, dfix_ref, st2_ref, s22_ref,
                        xv3_ref, est2_ref, c1, cb1,
                        scal_ref[2, tid], scal_ref[3, tid], iota)

    h = _lr(jnp.dot(feats2, o1_ref[...], preferred_element_type=f32)
            + ob1_ref[...])
    h = _lr(jnp.dot(h, o2_ref[...], preferred_element_type=f32) + ob2_ref[...])
    h = _lr(jnp.dot(h, o3_ref[...], preferred_element_type=f32) + ob3_ref[...])
    h = _lr(jnp.dot(h, o4_ref[...], preferred_element_type=f32) + ob4_ref[...])
    out_ref[...] = h


def kernel(x_pfc, x_vtx, batch_pfc, batch_vtx,
           P1, pb1, P2, pb2, V1, vb1, V2, vb2, C1, cb1,
           O1, ob1, O2, ob2, O3, ob3, O4, ob4):
    f32 = jnp.float32
    i32 = jnp.int32
    n_pfc = x_pfc.shape[0]
    n_vtx = x_vtx.shape[0]
    nc1 = n_pfc // CHUNK
    nc2 = n_vtx // CHUNK
    ntiles = n_pfc // TILE

    xe = _encode(x_pfc, P1, pb1.reshape(1, -1), P2, pb2.reshape(1, -1))
    xv = _encode(x_vtx, V1, vb1.reshape(1, -1), V2, vb2.reshape(1, -1))

    st1 = xe.T.reshape(HID, nc1, CHUNK).transpose(1, 0, 2)
    s21 = jnp.sum(xe ** 2, axis=1).reshape(nc1, 1, CHUNK)
    xe3 = xe.reshape(nc1, CHUNK, HID)
    st2 = xv.T.reshape(HID, nc2, CHUNK).transpose(1, 0, 2)
    s22 = jnp.sum(xv ** 2, axis=1).reshape(nc2, 1, CHUNK)
    xv3 = xv.reshape(nc2, CHUNK, HID)

    et_pfc = jax.nn.one_hot(batch_pfc, NB, dtype=f32)
    est1 = et_pfc.T.reshape(NB, nc1, CHUNK).transpose(1, 0, 2)
    est2 = (jax.nn.one_hot(batch_vtx, NB, dtype=f32)
            .T.reshape(NB, nc2, CHUNK).transpose(1, 0, 2))

    bf = batch_pfc[::TILE]
    bl = batch_pfc[TILE - 1::TILE]
    wsp = jnp.searchsorted(batch_pfc, bf, side='left').astype(i32)
    wep = jnp.searchsorted(batch_pfc, bl, side='right').astype(i32)
    wsv = jnp.searchsorted(batch_vtx, bf, side='left').astype(i32)
    wev = jnp.searchsorted(batch_vtx, bl, side='right').astype(i32)
    c0p = wsp // CHUNK
    nwp = -((-wep) // CHUNK) - c0p
    c0v = wsv // CHUNK
    nwv = -((-wev) // CHUNK) - c0v
    scal = jnp.stack([c0p, nwp, c0v, nwv])

    full3 = lambda a: pl.BlockSpec(a.shape, lambda t, s: (0, 0, 0))
    full2 = lambda a: pl.BlockSpec(a.shape, lambda t, s: (0, 0))

    grid_spec = pltpu.PrefetchScalarGridSpec(
        num_scalar_prefetch=1,
        grid=(ntiles,),
        in_specs=[
            pl.BlockSpec((TILE, HID), lambda t, s: (t, 0)),
            pl.BlockSpec((TILE, NB), lambda t, s: (t, 0)),
            full3(st1), full3(s21), full3(xe3), full3(est1),
            full3(st2), full3(s22), full3(xv3), full3(est2),
            full2(C1), pl.BlockSpec((1, HID), lambda t, s: (0, 0)),
            full2(O1), pl.BlockSpec((1, 64), lambda t, s: (0, 0)),
            full2(O2), pl.BlockSpec((1, 32), lambda t, s: (0, 0)),
            full2(O3), pl.BlockSpec((1, 4), lambda t, s: (0, 0)),
            full2(O4), pl.BlockSpec((1, 1), lambda t, s: (0, 0)),
        ],
        out_specs=pl.BlockSpec((TILE, 1), lambda t, s: (t, 0)),
        scratch_shapes=[
            pltpu.VMEM((nc1, TILE, CHUNK), f32),
            pltpu.VMEM((TILE, CHUNK), f32),
        ],
    )
    out = pl.pallas_call(
        _net_kernel,
        grid_spec=grid_spec,
        out_shape=jax.ShapeDtypeStruct((n_pfc, 1), f32),
    )(scal, xe, et_pfc, st1, s21, xe3, est1, st2, s22, xv3, est2,
      C1, cb1.reshape(1, -1),
      O1, ob1.reshape(1, -1), O2, ob2.reshape(1, -1),
      O3, ob3.reshape(1, -1), O4, ob4.reshape(1, -1))
    return (out, batch_pfc)


# speedup vs baseline: 5.1182x; 1.0023x over previous
import functools

import jax
import jax.numpy as jnp
from jax.experimental import pallas as pl
from jax.experimental.pallas import tpu as pltpu

HID = 32
K = 16
NB = 16
CHUNK = 512
TILE = 256
BIG = 3e38
MASKVAL = 1e10
IDXBIG = 1e9
HIGHEST = jax.lax.Precision.HIGHEST


def _lr(x):
    return jnp.where(x > 0, x, 0.01 * x)


def _encode_kernel(x_ref, w1_ref, b1_ref, w2_ref, b2_ref, xe_ref):
    h = _lr(jnp.dot(x_ref[...], w1_ref[...],
                    preferred_element_type=jnp.float32) + b1_ref[...])
    xe_ref[...] = _lr(jnp.dot(h, w2_ref[...],
                              preferred_element_type=jnp.float32) + b2_ref[...])


def _encode(x, w1, b1, w2, b2):
    n, din = x.shape
    nc = n // CHUNK
    return pl.pallas_call(
        _encode_kernel,
        grid=(nc,),
        in_specs=[
            pl.BlockSpec((CHUNK, din), lambda r: (r, 0)),
            pl.BlockSpec((din, HID), lambda r: (0, 0)),
            pl.BlockSpec((1, HID), lambda r: (0, 0)),
            pl.BlockSpec((HID, HID), lambda r: (0, 0)),
            pl.BlockSpec((1, HID), lambda r: (0, 0)),
        ],
        out_specs=pl.BlockSpec((CHUNK, HID), lambda r: (r, 0)),
        out_shape=jax.ShapeDtypeStruct((n, HID), jnp.float32),
        compiler_params=pltpu.CompilerParams(
            dimension_semantics=("parallel",)),
    )(x, w1, b1, w2, b2)


def _edge_conv(t, et, d_ref, dfix_ref, st_ref, s2_ref, xe_ref, est_ref,
               c1, cb1, c0, nw, iota):
    f32 = jnp.float32
    base = jax.lax.convert_element_type(c0, f32) * CHUNK

    same0 = jnp.dot(et, est_ref[0], preferred_element_type=f32)
    dfix_ref[...] = jnp.where(same0 > 0.5, BIG, MASKVAL)

    def build(j, c):
        r = c0 + j
        m = jnp.dot(t, st_ref[r], preferred_element_type=f32)
        d = -2.0 * m + s2_ref[r]
        same = jnp.dot(et, est_ref[r], preferred_element_type=f32)
        d_ref[j] = jnp.where(same > 0.5, d, BIG)
        return c

    jax.lax.fori_loop(0, nw, build, 0)

    def one_round(_, acc):
        def scan_chunk(j, mi):
            m, i = mi
            dd = d_ref[j]
            col = iota + base + jax.lax.convert_element_type(j, f32) * CHUNK
            mr = jnp.min(dd, axis=1, keepdims=True)
            ir = jnp.min(jnp.where(dd == mr, col, IDXBIG), axis=1,
                         keepdims=True)
            tie = mr == m
            i = jnp.where(mr < m, ir, jnp.where(tie, jnp.minimum(i, ir), i))
            m = jnp.minimum(m, mr)
            return m, i

        m0 = jnp.full((TILE, 1), BIG, f32)
        i0 = jnp.full((TILE, 1), IDXBIG, f32)
        msel, isel = jax.lax.fori_loop(0, nw, scan_chunk, (m0, i0))
        dd0 = dfix_ref[...]
        mr0 = jnp.min(dd0, axis=1, keepdims=True)
        ir0 = jnp.min(jnp.where(dd0 == mr0, iota, IDXBIG), axis=1,
                      keepdims=True)
        tie0 = mr0 == msel
        isel = jnp.where(mr0 < msel, ir0,
                         jnp.where(tie0, jnp.minimum(isel, ir0), isel))
        msel = jnp.minimum(msel, mr0)
        fromfix = msel == MASKVAL

        def gather_chunk(j, g):
            dd = d_ref[j]
            col = iota + base + jax.lax.convert_element_type(j, f32) * CHUNK
            hit = (col == isel) & jnp.logical_not(fromfix)
            d_ref[j] = jnp.where(hit, BIG, dd)
            return g + jnp.dot(hit.astype(f32), xe_ref[c0 + j],
                               preferred_element_type=f32, precision=HIGHEST)

        xj = jax.lax.fori_loop(0, nw, gather_chunk,
                               jnp.zeros((TILE, HID), f32))
        hit0 = (iota == isel) & fromfix
        dfix_ref[...] = jnp.where(hit0, BIG, dfix_ref[...])
        xj = xj + jnp.dot(hit0.astype(f32), xe_ref[0],
                          preferred_element_type=f32, precision=HIGHEST)

        mm = jnp.concatenate([t, xj - t], axis=1)
        msg = _lr(jnp.dot(mm, c1, preferred_element_type=f32) + cb1)
        return jnp.maximum(acc, msg)

    acc0 = jnp.full((TILE, HID), -BIG, f32)
    return jax.lax.fori_loop(0, K, one_round, acc0)


def _net_kernel(scal_ref,
                xe_ref, et_ref,
                st1_ref, s21_ref, xe3_ref, est1_ref,
                st2_ref, s22_ref, xv3_ref, est2_ref,
                c1_ref, cb1_ref,
                o1_ref, ob1_ref, o2_ref, ob2_ref,
                o3_ref, ob3_ref, o4_ref, ob4_ref,
                out_ref, d_ref, dfix_ref):
    f32 = jnp.float32
    tid = pl.program_id(0)
    iota = jax.lax.broadcasted_iota(jnp.int32, (TILE, CHUNK), 1).astype(f32)
    et = et_ref[...]
    c1 = c1_ref[...]
    cb1 = cb1_ref[...]

    feats1 = _edge_conv(xe_ref[...], et, d_ref, dfix_ref, st1_ref, s21_ref,
                        xe3_ref, est1_ref, c1, cb1,
                        scal_ref[0, tid], scal_ref[1, tid], iota)
    feats2 = _edge_conv(feats1, et, d_ref, dfix_ref, st2_ref, s22_ref,
                        xv3_ref, est2_ref, c1, cb1,
                        scal_ref[2, tid], scal_ref[3, tid], iota)

    h = _lr(jnp.dot(feats2, o1_ref[...], preferred_element_type=f32)
            + ob1_ref[...])
    h = _lr(jnp.dot(h, o2_ref[...], preferred_element_type=f32) + ob2_ref[...])
    h = _lr(jnp.dot(h, o3_ref[...], preferred_element_type=f32) + ob3_ref[...])
    h = _lr(jnp.dot(h, o4_ref[...], preferred_element_type=f32) + ob4_ref[...])
    out_ref[...] = h


def kernel(x_pfc, x_vtx, batch_pfc, batch_vtx,
           P1, pb1, P2, pb2, V1, vb1, V2, vb2, C1, cb1,
           O1, ob1, O2, ob2, O3, ob3, O4, ob4):
    f32 = jnp.float32
    i32 = jnp.int32
    n_pfc = x_pfc.shape[0]
    n_vtx = x_vtx.shape[0]
    nc1 = n_pfc // CHUNK
    nc2 = n_vtx // CHUNK
    ntiles = n_pfc // TILE

    xe = _encode(x_pfc, P1, pb1.reshape(1, -1), P2, pb2.reshape(1, -1))
    xv = _encode(x_vtx, V1, vb1.reshape(1, -1), V2, vb2.reshape(1, -1))

    st1 = xe.T.reshape(HID, nc1, CHUNK).transpose(1, 0, 2)
    s21 = jnp.sum(xe ** 2, axis=1).reshape(nc1, 1, CHUNK)
    xe3 = xe.reshape(nc1, CHUNK, HID)
    st2 = xv.T.reshape(HID, nc2, CHUNK).transpose(1, 0, 2)
    s22 = jnp.sum(xv ** 2, axis=1).reshape(nc2, 1, CHUNK)
    xv3 = xv.reshape(nc2, CHUNK, HID)

    et_pfc = jax.nn.one_hot(batch_pfc, NB, dtype=f32)
    est1 = et_pfc.T.reshape(NB, nc1, CHUNK).transpose(1, 0, 2)
    est2 = (jax.nn.one_hot(batch_vtx, NB, dtype=f32)
            .T.reshape(NB, nc2, CHUNK).transpose(1, 0, 2))

    bf = batch_pfc[::TILE]
    bl = batch_pfc[TILE - 1::TILE]
    wsp = jnp.searchsorted(batch_pfc, bf, side='left').astype(i32)
    wep = jnp.searchsorted(batch_pfc, bl, side='right').astype(i32)
    wsv = jnp.searchsorted(batch_vtx, bf, side='left').astype(i32)
    wev = jnp.searchsorted(batch_vtx, bl, side='right').astype(i32)
    c0p = wsp // CHUNK
    nwp = -((-wep) // CHUNK) - c0p
    c0v = wsv // CHUNK
    nwv = -((-wev) // CHUNK) - c0v
    scal = jnp.stack([c0p, nwp, c0v, nwv])

    full3 = lambda a: pl.BlockSpec(a.shape, lambda t, s: (0, 0, 0))
    full2 = lambda a: pl.BlockSpec(a.shape, lambda t, s: (0, 0))

    grid_spec = pltpu.PrefetchScalarGridSpec(
        num_scalar_prefetch=1,
        grid=(ntiles,),
        in_specs=[
            pl.BlockSpec((TILE, HID), lambda t, s: (t, 0)),
            pl.BlockSpec((TILE, NB), lambda t, s: (t, 0)),
            full3(st1), full3(s21), full3(xe3), full3(est1),
            full3(st2), full3(s22), full3(xv3), full3(est2),
            full2(C1), pl.BlockSpec((1, HID), lambda t, s: (0, 0)),
            full2(O1), pl.BlockSpec((1, 64), lambda t, s: (0, 0)),
            full2(O2), pl.BlockSpec((1, 32), lambda t, s: (0, 0)),
            full2(O3), pl.BlockSpec((1, 4), lambda t, s: (0, 0)),
            full2(O4), pl.BlockSpec((1, 1), lambda t, s: (0, 0)),
        ],
        out_specs=pl.BlockSpec((TILE, 1), lambda t, s: (t, 0)),
        scratch_shapes=[
            pltpu.VMEM((nc1, TILE, CHUNK), f32),
            pltpu.VMEM((TILE, CHUNK), f32),
        ],
    )
    out = pl.pallas_call(
        _net_kernel,
        grid_spec=grid_spec,
        out_shape=jax.ShapeDtypeStruct((n_pfc, 1), f32),
        compiler_params=pltpu.CompilerParams(
            dimension_semantics=("parallel",)),
    )(scal, xe, et_pfc, st1, s21, xe3, est1, st2, s22, xv3, est2,
      C1, cb1.reshape(1, -1),
      O1, ob1.reshape(1, -1), O2, ob2.reshape(1, -1),
      O3, ob3.reshape(1, -1), O4, ob4.reshape(1, -1))
    return (out, batch_pfc)


# speedup vs baseline: 5.8900x; 1.1508x over previous
import functools

import jax
import jax.numpy as jnp
from jax.experimental import pallas as pl
from jax.experimental.pallas import tpu as pltpu

HID = 32
K = 16
NB = 16
CHUNK = 512
TILE = 256
BIG = 3e38
MASKVAL = 1e10
IDXBIG = 1e9
FIX = 128
HIGHEST = jax.lax.Precision.HIGHEST


def _lr(x):
    return jnp.where(x > 0, x, 0.01 * x)


def _encode_kernel(x_ref, w1_ref, b1_ref, w2_ref, b2_ref, xe_ref):
    h = _lr(jnp.dot(x_ref[...], w1_ref[...],
                    preferred_element_type=jnp.float32) + b1_ref[...])
    xe_ref[...] = _lr(jnp.dot(h, w2_ref[...],
                              preferred_element_type=jnp.float32) + b2_ref[...])


def _encode(x, w1, b1, w2, b2):
    n, din = x.shape
    nc = n // CHUNK
    return pl.pallas_call(
        _encode_kernel,
        grid=(nc,),
        in_specs=[
            pl.BlockSpec((CHUNK, din), lambda r: (r, 0)),
            pl.BlockSpec((din, HID), lambda r: (0, 0)),
            pl.BlockSpec((1, HID), lambda r: (0, 0)),
            pl.BlockSpec((HID, HID), lambda r: (0, 0)),
            pl.BlockSpec((1, HID), lambda r: (0, 0)),
        ],
        out_specs=pl.BlockSpec((CHUNK, HID), lambda r: (r, 0)),
        out_shape=jax.ShapeDtypeStruct((n, HID), jnp.float32),
        compiler_params=pltpu.CompilerParams(
            dimension_semantics=("parallel",)),
    )(x, w1, b1, w2, b2)


def _edge_conv(t, s2t, et, d_ref, dfix_ref, st_ref, s2_ref, xe_ref, est_ref,
               c1, cb1, c0, nw, iota):
    f32 = jnp.float32
    i32 = jnp.int32
    bf16 = jnp.bfloat16
    base = jax.lax.convert_element_type(c0, f32) * CHUNK
    iota_f = iota[:, :FIX]
    tb = t.astype(bf16)
    c1b = c1.astype(bf16)

    same0 = jnp.dot(et, est_ref[0][:, :FIX], preferred_element_type=f32)
    dfix_ref[...] = jnp.where(same0 > 0.5, BIG, MASKVAL)

    def build(j, c):
        r = c0 + j
        m = jnp.dot(tb, st_ref[r].astype(bf16), preferred_element_type=f32)
        d = (s2t - 2.0 * m) + s2_ref[r]
        same = jnp.dot(et, est_ref[r], preferred_element_type=f32)
        d_ref[j] = jnp.where(same > 0.5, d, BIG)
        return c

    jax.lax.fori_loop(0, nw, build, 0)

    def one_round(_, acc):
        def scan_chunk(j, mi):
            m, i = mi
            dd = d_ref[j]
            col = iota + base + jax.lax.convert_element_type(j, f32) * CHUNK
            mr = jnp.min(dd, axis=1, keepdims=True)
            ir = jnp.min(jnp.where(dd == mr, col, IDXBIG), axis=1,
                         keepdims=True)
            tie = mr == m
            i = jnp.where(mr < m, ir, jnp.where(tie, jnp.minimum(i, ir), i))
            m = jnp.minimum(m, mr)
            return m, i

        m0 = jnp.full((TILE, 1), BIG, f32)
        i0 = jnp.full((TILE, 1), IDXBIG, f32)
        msel, isel = jax.lax.fori_loop(0, nw, scan_chunk, (m0, i0))
        dd0 = dfix_ref[...]
        mr0 = jnp.min(dd0, axis=1, keepdims=True)
        ir0 = jnp.min(jnp.where(dd0 == mr0, iota_f, IDXBIG), axis=1,
                      keepdims=True)
        tie0 = mr0 == msel
        isel = jnp.where(mr0 < msel, ir0,
                         jnp.where(tie0, jnp.minimum(isel, ir0), isel))
        msel = jnp.minimum(msel, mr0)
        fromfix = msel == MASKVAL

        def gather_chunk(j, g):
            dd = d_ref[j]
            col = iota + base + jax.lax.convert_element_type(j, f32) * CHUNK
            hit = col == isel
            d_ref[j] = jnp.where(hit, BIG, dd)
            ghit = hit & jnp.logical_not(fromfix)
            return g + jnp.dot(ghit.astype(f32), xe_ref[c0 + j],
                               preferred_element_type=f32, precision=HIGHEST)

        g3 = jax.lax.fori_loop(0, nw, gather_chunk,
                               jnp.zeros((TILE, HID), f32))
        hit0 = (iota_f == isel) & fromfix
        dfix_ref[...] = jnp.where(hit0, BIG, dfix_ref[...])
        xj = g3 + jnp.dot(hit0.astype(f32), xe_ref[0, :FIX],
                          preferred_element_type=f32, precision=HIGHEST)

        mm = jnp.concatenate([t, xj - t], axis=1).astype(bf16)
        msg = _lr(jnp.dot(mm, c1b, preferred_element_type=f32) + cb1)
        return jnp.maximum(acc, msg)

    acc0 = jnp.full((TILE, HID), -BIG, f32)
    return jax.lax.fori_loop(0, K, one_round, acc0)


def _conv1_kernel(scal_ref,
                  xe_ref, s2t_ref, et_ref,
                  st1_ref, s21_ref, xe3_ref, est1_ref,
                  c1_ref, cb1_ref,
                  f1_ref, d_ref, dfix_ref):
    f32 = jnp.float32
    tid = pl.program_id(0)
    iota = jax.lax.broadcasted_iota(jnp.int32, (TILE, CHUNK), 1).astype(f32)
    f1_ref[...] = _edge_conv(xe_ref[...], s2t_ref[...], et_ref[...], d_ref,
                             dfix_ref, st1_ref, s21_ref, xe3_ref, est1_ref,
                             c1_ref[...], cb1_ref[...],
                             scal_ref[0, tid], scal_ref[1, tid], iota)


def _conv2_kernel(scal_ref,
                  f1_ref, s2t_ref, et_ref,
                  st2_ref, s22_ref, xv3_ref, est2_ref,
                  c1_ref, cb1_ref,
                  o1_ref, ob1_ref, o2_ref, ob2_ref,
                  o3_ref, ob3_ref, o4_ref, ob4_ref,
                  out_ref, d_ref, dfix_ref):
    f32 = jnp.float32
    tid = pl.program_id(0)
    iota = jax.lax.broadcasted_iota(jnp.int32, (TILE, CHUNK), 1).astype(f32)
    feats2 = _edge_conv(f1_ref[...], s2t_ref[...], et_ref[...], d_ref,
                        dfix_ref, st2_ref, s22_ref, xv3_ref, est2_ref,
                        c1_ref[...], cb1_ref[...],
                        scal_ref[2, tid], scal_ref[3, tid], iota)

    bf16 = jnp.bfloat16
    h = _lr(jnp.dot(feats2.astype(bf16), o1_ref[...].astype(bf16),
                    preferred_element_type=f32) + ob1_ref[...])
    h = _lr(jnp.dot(h.astype(bf16), o2_ref[...].astype(bf16),
                    preferred_element_type=f32) + ob2_ref[...])
    h = _lr(jnp.dot(h.astype(bf16), o3_ref[...].astype(bf16),
                    preferred_element_type=f32) + ob3_ref[...])
    h = _lr(jnp.dot(h.astype(bf16), o4_ref[...].astype(bf16),
                    preferred_element_type=f32) + ob4_ref[...])
    out_ref[...] = h


def _split3(x):
    f32 = jnp.float32
    p1 = x.astype(jnp.bfloat16).astype(f32)
    r = x - p1
    p2 = r.astype(jnp.bfloat16).astype(f32)
    p3 = r - p2
    return jnp.concatenate([p1, p2, p3], axis=1)


def kernel(x_pfc, x_vtx, batch_pfc, batch_vtx,
           P1, pb1, P2, pb2, V1, vb1, V2, vb2, C1, cb1,
           O1, ob1, O2, ob2, O3, ob3, O4, ob4):
    f32 = jnp.float32
    i32 = jnp.int32
    n_pfc = x_pfc.shape[0]
    n_vtx = x_vtx.shape[0]
    nc1 = n_pfc // CHUNK
    nc2 = n_vtx // CHUNK
    ntiles = n_pfc // TILE

    xe = _encode(x_pfc, P1, pb1.reshape(1, -1), P2, pb2.reshape(1, -1))
    xv = _encode(x_vtx, V1, vb1.reshape(1, -1), V2, vb2.reshape(1, -1))

    st1 = xe.T.reshape(HID, nc1, CHUNK).transpose(1, 0, 2)
    s2p = jnp.sum(xe ** 2, axis=1)
    s21 = s2p.reshape(nc1, 1, CHUNK)
    xe3 = xe.reshape(nc1, CHUNK, HID)
    st2 = xv.T.reshape(HID, nc2, CHUNK).transpose(1, 0, 2)
    s22 = jnp.sum(xv ** 2, axis=1).reshape(nc2, 1, CHUNK)
    xv3 = xv.reshape(nc2, CHUNK, HID)

    et_pfc = jax.nn.one_hot(batch_pfc, NB, dtype=f32)
    est1 = et_pfc.T.reshape(NB, nc1, CHUNK).transpose(1, 0, 2)
    est2 = (jax.nn.one_hot(batch_vtx, NB, dtype=f32)
            .T.reshape(NB, nc2, CHUNK).transpose(1, 0, 2))

    bf = batch_pfc[::TILE]
    bl = batch_pfc[TILE - 1::TILE]
    wsp = jnp.searchsorted(batch_pfc, bf, side='left').astype(i32)
    wep = jnp.searchsorted(batch_pfc, bl, side='right').astype(i32)
    wsv = jnp.searchsorted(batch_vtx, bf, side='left').astype(i32)
    wev = jnp.searchsorted(batch_vtx, bl, side='right').astype(i32)
    c0p = wsp // CHUNK
    nwp = -((-wep) // CHUNK) - c0p
    c0v = wsv // CHUNK
    nwv = -((-wev) // CHUNK) - c0v
    scal = jnp.stack([c0p, nwp, c0v, nwv])

    full3 = lambda a: pl.BlockSpec(a.shape, lambda t, s: (0, 0, 0))
    full2 = lambda a: pl.BlockSpec(a.shape, lambda t, s: (0, 0))
    tile2 = lambda w: pl.BlockSpec((TILE, w), lambda t, s: (t, 0))
    cparams = pltpu.CompilerParams(dimension_semantics=("parallel",))

    s2p_col = s2p[:, None]
    feats1 = pl.pallas_call(
        _conv1_kernel,
        grid_spec=pltpu.PrefetchScalarGridSpec(
            num_scalar_prefetch=1,
            grid=(ntiles,),
            in_specs=[
                tile2(HID), tile2(1), tile2(NB),
                full3(st1), full3(s21), full3(xe3), full3(est1),
                full2(C1), pl.BlockSpec((1, HID), lambda t, s: (0, 0)),
            ],
            out_specs=tile2(HID),
            scratch_shapes=[
                pltpu.VMEM((nc1, TILE, CHUNK), f32),
                pltpu.VMEM((TILE, FIX), f32),
            ],
        ),
        out_shape=jax.ShapeDtypeStruct((n_pfc, HID), f32),
        compiler_params=cparams,
    )(scal, xe, s2p_col, et_pfc, st1, s21, xe3, est1, C1, cb1.reshape(1, -1))

    s2f_col = jnp.sum(feats1 ** 2, axis=1)[:, None]

    out = pl.pallas_call(
        _conv2_kernel,
        grid_spec=pltpu.PrefetchScalarGridSpec(
            num_scalar_prefetch=1,
            grid=(ntiles,),
            in_specs=[
                tile2(HID), tile2(1), tile2(NB),
                full3(st2), full3(s22), full3(xv3), full3(est2),
                full2(C1), pl.BlockSpec((1, HID), lambda t, s: (0, 0)),
                full2(O1), pl.BlockSpec((1, 64), lambda t, s: (0, 0)),
                full2(O2), pl.BlockSpec((1, 32), lambda t, s: (0, 0)),
                full2(O3), pl.BlockSpec((1, 4), lambda t, s: (0, 0)),
                full2(O4), pl.BlockSpec((1, 1), lambda t, s: (0, 0)),
            ],
            out_specs=tile2(1),
            scratch_shapes=[
                pltpu.VMEM((nc2, TILE, CHUNK), f32),
                pltpu.VMEM((TILE, FIX), f32),
            ],
        ),
        out_shape=jax.ShapeDtypeStruct((n_pfc, 1), f32),
        compiler_params=cparams,
    )(scal, feats1, s2f_col, et_pfc, st2, s22, xv3, est2,
      C1, cb1.reshape(1, -1),
      O1, ob1.reshape(1, -1), O2, ob2.reshape(1, -1),
      O3, ob3.reshape(1, -1), O4, ob4.reshape(1, -1))
    return (out, batch_pfc)
